# Initial kernel scaffold; baseline (speedup 1.0000x reference)
import jax
import jax.numpy as jnp
from jax.experimental import pallas as pl


def kernel(feat_v, feat_u, neigh_v, neigh_u, target, Wv1, bv1, Wu1, bu1, a1, Wv2, bv2, Wu2, bu2, a2, Wv3, bv3, Wu3, bu3, Wd):
    raise NotImplementedError("write your pallas kernel here")


# speedup vs baseline: 3.5976x; 3.5976x over previous
import functools

import jax
import jax.numpy as jnp
from jax import lax
from jax.experimental import pallas as pl
from jax.experimental.pallas import tpu as pltpu
from jax.experimental.pallas import tpu_sc as plsc

NV = 4096
D = 256
DEG = 16

NC = 2
NS = 16
L = 16
NW = NC * NS
NPW = NV // NW
CH = 8
RPC = CH * DEG
NCHUNK = NPW // CH


@functools.lru_cache(maxsize=None)
def _make_sc_agg(with_addend: bool, scale: float):
  mesh = plsc.VectorSubcoreMesh(
      core_axis_name="c", subcore_axis_name="s", num_cores=NC, num_subcores=NS)
  out_type = (jax.ShapeDtypeStruct((NV, D), jnp.float32),
              jax.ShapeDtypeStruct((NV, D), jnp.float32))
  scratch = [
      pltpu.VMEM((RPC,), jnp.int32),
      pltpu.VMEM((RPC, D), jnp.float32),
      pltpu.VMEM((CH, D), jnp.float32),
      pltpu.SemaphoreType.DMA,
  ]
  if with_addend:
    scratch.append(pltpu.VMEM((CH, D), jnp.float32))

  def body(*refs):
    if with_addend:
      (ta, ia, tb, ib, ada, adb, outa, outb,
       idx_v, rows_v, out_v, sem, add_v) = refs
      sides = ((ta, ia, ada, outa), (tb, ib, adb, outb))
    else:
      ta, ia, tb, ib, outa, outb, idx_v, rows_v, out_v, sem = refs
      add_v = None
      sides = ((ta, ia, None, outa), (tb, ib, None, outb))
    wid = lax.axis_index("s") * NC + lax.axis_index("c")
    node0 = wid * NPW

    for table, idx, addend, out in sides:
      def chunk_body(k, _, table=table, idx=idx, addend=addend, out=out):
        nbase = node0 + k * CH
        pltpu.sync_copy(idx.at[pl.ds(nbase * DEG, RPC)], idx_v)
        pltpu.async_copy(table.at[idx_v], rows_v, sem).wait()
        if addend is not None:
          pltpu.sync_copy(addend.at[pl.ds(nbase, CH)], add_v)

        def lane_body(l, _):
          sl = pl.ds(l * L, L)
          for c in range(CH):
            acc = rows_v[c * DEG, sl]
            for r in range(1, DEG):
              acc = acc + rows_v[c * DEG + r, sl]
            if addend is not None:
              acc = acc + add_v[c, sl]
            out_v[c, sl] = acc * scale
          return 0

        lax.fori_loop(0, D // L, lane_body, 0)
        pltpu.sync_copy(out_v, out.at[pl.ds(nbase, CH)])
        return 0

      lax.fori_loop(0, NCHUNK, chunk_body, 0)

  return pl.kernel(body, out_type=out_type, mesh=mesh, scratch_types=scratch)


def _sc_agg_mean(*args):
  return _make_sc_agg(False, 1.0 / DEG)(*args)


def _sc_agg_self(*args):
  return _make_sc_agg(True, 1.0 / (DEG + 1.0))(*args)


BM = 512
NBLK = NV // BM


def _prelu(y, a):
  return jnp.where(y >= 0, y, a * y)


def _stage12_body(has_wd, xv, xu, wv, wu, bv, bu, a_ref, *outs):
  a = a_ref[0]
  ev = _prelu(jnp.dot(xv[...], wv[...], preferred_element_type=jnp.float32)
              + bv[...], a)
  eu = _prelu(jnp.dot(xu[...], wu[...], preferred_element_type=jnp.float32)
              + bu[...], a)
  if has_wd:
    wd, ov, ou, oa = outs
    oa[...] = jnp.dot(ev, wd[...], preferred_element_type=jnp.float32)
  else:
    ov, ou = outs
  ov[...] = ev
  ou[...] = eu


def _row_spec():
  return pl.BlockSpec((BM, D), lambda i: (i, 0))


def _full_spec():
  return pl.BlockSpec((D, D), lambda i: (0, 0))


def _bias_spec():
  return pl.BlockSpec((1, D), lambda i: (0, 0))


_stage1_call = pl.pallas_call(
    functools.partial(_stage12_body, False),
    grid=(NBLK,),
    in_specs=[_row_spec(), _row_spec(), _full_spec(), _full_spec(),
              _bias_spec(), _bias_spec(),
              pl.BlockSpec(memory_space=pltpu.SMEM)],
    out_specs=[_row_spec(), _row_spec()],
    out_shape=[jax.ShapeDtypeStruct((NV, D), jnp.float32)] * 2,
)

_stage2_call = pl.pallas_call(
    functools.partial(_stage12_body, True),
    grid=(NBLK,),
    in_specs=[_row_spec(), _row_spec(), _full_spec(), _full_spec(),
              _bias_spec(), _bias_spec(),
              pl.BlockSpec(memory_space=pltpu.SMEM), _full_spec()],
    out_specs=[_row_spec(), _row_spec(), _row_spec()],
    out_shape=[jax.ShapeDtypeStruct((NV, D), jnp.float32)] * 3,
)


def _stage3_body(ev2, eu2, fv, fu, wv3, wu3, bv3, bu3, ov, ou):
  ov[...] = (jnp.dot(ev2[...], wv3[:D, :], preferred_element_type=jnp.float32)
             + jnp.dot(fv[...], wv3[D:, :], preferred_element_type=jnp.float32)
             + bv3[...])
  ou[...] = (jnp.dot(eu2[...], wu3[:D, :], preferred_element_type=jnp.float32)
             + jnp.dot(fu[...], wu3[D:, :], preferred_element_type=jnp.float32)
             + bu3[...])


_stage3_call = pl.pallas_call(
    _stage3_body,
    grid=(NBLK,),
    in_specs=[_row_spec(), _row_spec(), _row_spec(), _row_spec(),
              pl.BlockSpec((2 * D, D), lambda i: (0, 0)),
              pl.BlockSpec((2 * D, D), lambda i: (0, 0)),
              _bias_spec(), _bias_spec()],
    out_specs=[_row_spec(), _row_spec()],
    out_shape=[jax.ShapeDtypeStruct((NV, D), jnp.float32)] * 2,
)


LBM = 512
LBN = 1024
LNI = NV // LBM
LNJ = NV // LBN


def _loss_body(a_ref, u_ref, t_ref, o_ref, acc_ref):
  i = pl.program_id(0)
  j = pl.program_id(1)

  @pl.when((i == 0) & (j == 0))
  def _():
    acc_ref[0] = 0.0
    acc_ref[1] = 0.0
    acc_ref[2] = 0.0
    acc_ref[3] = 0.0

  logit = lax.dot_general(a_ref[...], u_ref[...], (((1,), (1,)), ((), ())),
                          preferred_element_type=jnp.float32)
  t = t_ref[...].astype(jnp.float32)
  sp = jnp.maximum(logit, 0.0) + jnp.log1p(jnp.exp(-jnp.abs(logit)))
  acc_ref[0] += jnp.sum(sp)
  acc_ref[1] += jnp.sum(t * (sp - logit))
  acc_ref[2] += jnp.sum(t * sp)
  acc_ref[3] += jnp.sum(t)

  @pl.when((i == LNI - 1) & (j == LNJ - 1))
  def _():
    n = float(NV) * float(NV)
    s = acc_ref[3]
    norm = n / (n - s)
    pw = (n - s) / s
    val = (norm / n) * (acc_ref[0] + pw * acc_ref[1] - acc_ref[2])
    o_ref[...] = jnp.reshape(val, (1, 1))


_loss_call = pl.pallas_call(
    _loss_body,
    grid=(LNI, LNJ),
    in_specs=[pl.BlockSpec((LBM, D), lambda i, j: (i, 0)),
              pl.BlockSpec((LBN, D), lambda i, j: (j, 0)),
              pl.BlockSpec((LBM, LBN), lambda i, j: (i, j))],
    out_specs=pl.BlockSpec((1, 1), lambda i, j: (0, 0)),
    out_shape=jax.ShapeDtypeStruct((1, 1), jnp.float32),
    scratch_shapes=[pltpu.SMEM((4,), jnp.float32)],
    compiler_params=pltpu.CompilerParams(
        dimension_semantics=("arbitrary", "arbitrary")),
)


def kernel(feat_v, feat_u, neigh_v, neigh_u, target,
           Wv1, bv1, Wu1, bu1, a1, Wv2, bv2, Wu2, bu2, a2,
           Wv3, bv3, Wu3, bu3, Wd):
  iv = neigh_v.reshape(-1)
  iu = neigh_u.reshape(-1)

  aggv1, aggu1 = _sc_agg_mean(feat_u, iv, feat_v, iu)
  ve1, ue1 = _stage1_call(aggv1, aggu1, Wv1, Wu1,
                          bv1.reshape(1, D), bu1.reshape(1, D),
                          a1.reshape(1))
  aggv2, aggu2 = _sc_agg_mean(ue1, iv, ve1, iu)
  ve2, ue2, A = _stage2_call(aggv2, aggu2, Wv2, Wu2,
                             bv2.reshape(1, D), bu2.reshape(1, D),
                             a2.reshape(1), Wd)
  ve3, ue3 = _stage3_call(ve2, ue2, feat_v, feat_u, Wv3, Wu3,
                          bv3.reshape(1, D), bu3.reshape(1, D))
  sv, su = _sc_agg_self(ue3, iv, ve3, iu, ve3, ue3)
  loss = _loss_call(A, ue2, target)[0, 0]
  return ve2, ue2, sv, su, loss


# speedup vs baseline: 6.0264x; 1.6751x over previous
import functools

import jax
import jax.numpy as jnp
from jax import lax
from jax.experimental import pallas as pl
from jax.experimental.pallas import tpu as pltpu
from jax.experimental.pallas import tpu_sc as plsc

NV = 4096
D = 256
DEG = 16

NC = 2
NS = 16
L = 16
NW = NC * NS
NPW = NV // NW
CH = 8
RPC = CH * DEG
NCHUNK = NPW // CH


@functools.lru_cache(maxsize=None)
def _make_sc_agg(with_addend: bool, scale: float):
  mesh = plsc.VectorSubcoreMesh(
      core_axis_name="c", subcore_axis_name="s", num_cores=NC, num_subcores=NS)
  out_type = (jax.ShapeDtypeStruct((NV, D), jnp.float32),
              jax.ShapeDtypeStruct((NV, D), jnp.float32))
  scratch = [
      pltpu.VMEM((NPW * DEG,), jnp.int32),
      pltpu.VMEM((2, RPC, D), jnp.float32),
      pltpu.VMEM((2, CH, D), jnp.float32),
      pltpu.SemaphoreType.DMA,
      pltpu.SemaphoreType.DMA,
      pltpu.SemaphoreType.DMA,
      pltpu.SemaphoreType.DMA,
  ]
  if with_addend:
    scratch.append(pltpu.VMEM((NPW, D), jnp.float32))

  def body(*refs):
    if with_addend:
      (ta, ia, tb, ib, ada, adb, outa, outb,
       idx_v, rows_v, out_v, gsem0, gsem1, osem0, osem1, add_v) = refs
      sides = ((ta, ia, ada, outa), (tb, ib, adb, outb))
    else:
      (ta, ia, tb, ib, outa, outb,
       idx_v, rows_v, out_v, gsem0, gsem1, osem0, osem1) = refs
      add_v = None
      sides = ((ta, ia, None, outa), (tb, ib, None, outb))
    gsems = (gsem0, gsem1)
    osems = (osem0, osem1)
    wid = lax.axis_index("s") * NC + lax.axis_index("c")
    node0 = wid * NPW

    for table, idx, addend, out in sides:
      pltpu.sync_copy(idx.at[pl.ds(node0 * DEG, NPW * DEG)], idx_v)
      if addend is not None:
        pltpu.sync_copy(addend.at[pl.ds(node0, NPW)], add_v)

      def start_gather(c, b, table=table):
        pltpu.async_copy(
            table.at[idx_v.at[pl.ds(c * RPC, RPC)]], rows_v.at[b], gsems[b])

      def reduce_chunk(c, b, addend=addend):
        def lane_body(l, _):
          sl = pl.ds(l * L, L)
          for cc in range(CH):
            acc = rows_v[b, cc * DEG, sl]
            for r in range(1, DEG):
              acc = acc + rows_v[b, cc * DEG + r, sl]
            if addend is not None:
              acc = acc + add_v[c * CH + cc, sl]
            out_v[b, cc, sl] = acc * scale
          return 0
        lax.fori_loop(0, D // L, lane_body, 0)

      start_gather(0, 0)

      def wait_gather(b, table=table):
        pltpu.make_async_copy(
            table.at[pl.ds(0, RPC)], rows_v.at[b], gsems[b]).wait()

      def wait_out(b, out=out):
        pltpu.make_async_copy(
            out_v.at[b], out.at[pl.ds(node0, CH)], osems[b]).wait()

      def pair_body(kk, _, out=out):
        for b in range(2):
          c = kk * 2 + b
          wait_gather(b)

          @pl.when(c + 1 < NCHUNK)
          def _():
            start_gather(c + 1, 1 - b)

          @pl.when(c >= 2)
          def _():
            wait_out(b)

          reduce_chunk(c, b)
          pltpu.async_copy(out_v.at[b],
                           out.at[pl.ds(node0 + c * CH, CH)], osems[b])
        return 0

      lax.fori_loop(0, NCHUNK // 2, pair_body, 0)
      for b in range(2):
        wait_out(b)

  return pl.kernel(body, out_type=out_type, mesh=mesh, scratch_types=scratch)


def _sc_agg_mean(*args):
  return _make_sc_agg(False, 1.0 / DEG)(*args)


def _sc_agg_self(*args):
  return _make_sc_agg(True, 1.0 / (DEG + 1.0))(*args)


BM = 512
NBLK = NV // BM


def _prelu(y, a):
  return jnp.where(y >= 0, y, a * y)


def _stage12_body(has_wd, xv, xu, wv, wu, bv, bu, a_ref, *outs):
  a = a_ref[0]
  ev = _prelu(jnp.dot(xv[...], wv[...], preferred_element_type=jnp.float32)
              + bv[...], a)
  eu = _prelu(jnp.dot(xu[...], wu[...], preferred_element_type=jnp.float32)
              + bu[...], a)
  if has_wd:
    wd, ov, ou, oa = outs
    oa[...] = jnp.dot(ev, wd[...], preferred_element_type=jnp.float32)
  else:
    ov, ou = outs
  ov[...] = ev
  ou[...] = eu


def _row_spec():
  return pl.BlockSpec((BM, D), lambda i: (i, 0))


def _full_spec():
  return pl.BlockSpec((D, D), lambda i: (0, 0))


def _bias_spec():
  return pl.BlockSpec((1, D), lambda i: (0, 0))


_stage1_call = pl.pallas_call(
    functools.partial(_stage12_body, False),
    grid=(NBLK,),
    in_specs=[_row_spec(), _row_spec(), _full_spec(), _full_spec(),
              _bias_spec(), _bias_spec(),
              pl.BlockSpec(memory_space=pltpu.SMEM)],
    out_specs=[_row_spec(), _row_spec()],
    out_shape=[jax.ShapeDtypeStruct((NV, D), jnp.float32)] * 2,
)

_stage2_call = pl.pallas_call(
    functools.partial(_stage12_body, True),
    grid=(NBLK,),
    in_specs=[_row_spec(), _row_spec(), _full_spec(), _full_spec(),
              _bias_spec(), _bias_spec(),
              pl.BlockSpec(memory_space=pltpu.SMEM), _full_spec()],
    out_specs=[_row_spec(), _row_spec(), _row_spec()],
    out_shape=[jax.ShapeDtypeStruct((NV, D), jnp.float32)] * 3,
)


def _stage3_body(ev2, eu2, fv, fu, wv3, wu3, bv3, bu3, ov, ou):
  ov[...] = (jnp.dot(ev2[...], wv3[:D, :], preferred_element_type=jnp.float32)
             + jnp.dot(fv[...], wv3[D:, :], preferred_element_type=jnp.float32)
             + bv3[...])
  ou[...] = (jnp.dot(eu2[...], wu3[:D, :], preferred_element_type=jnp.float32)
             + jnp.dot(fu[...], wu3[D:, :], preferred_element_type=jnp.float32)
             + bu3[...])


_stage3_call = pl.pallas_call(
    _stage3_body,
    grid=(NBLK,),
    in_specs=[_row_spec(), _row_spec(), _row_spec(), _row_spec(),
              pl.BlockSpec((2 * D, D), lambda i: (0, 0)),
              pl.BlockSpec((2 * D, D), lambda i: (0, 0)),
              _bias_spec(), _bias_spec()],
    out_specs=[_row_spec(), _row_spec()],
    out_shape=[jax.ShapeDtypeStruct((NV, D), jnp.float32)] * 2,
)


LBM = 512
LBN = 1024
LNI = NV // LBM
LNJ = NV // LBN


def _loss_body(a_ref, u_ref, t_ref, o_ref, acc_ref):
  i = pl.program_id(0)
  j = pl.program_id(1)

  @pl.when((i == 0) & (j == 0))
  def _():
    acc_ref[0] = 0.0
    acc_ref[1] = 0.0
    acc_ref[2] = 0.0
    acc_ref[3] = 0.0

  logit = lax.dot_general(a_ref[...], u_ref[...], (((1,), (1,)), ((), ())),
                          preferred_element_type=jnp.float32)
  t = t_ref[...].astype(jnp.float32)
  sp = jnp.maximum(logit, 0.0) + jnp.log1p(jnp.exp(-jnp.abs(logit)))
  acc_ref[0] += jnp.sum(sp)
  acc_ref[1] += jnp.sum(t * (sp - logit))
  acc_ref[2] += jnp.sum(t * sp)
  acc_ref[3] += jnp.sum(t)

  @pl.when((i == LNI - 1) & (j == LNJ - 1))
  def _():
    n = float(NV) * float(NV)
    s = acc_ref[3]
    norm = n / (n - s)
    pw = (n - s) / s
    val = (norm / n) * (acc_ref[0] + pw * acc_ref[1] - acc_ref[2])
    o_ref[...] = jnp.reshape(val, (1, 1))


_loss_call = pl.pallas_call(
    _loss_body,
    grid=(LNI, LNJ),
    in_specs=[pl.BlockSpec((LBM, D), lambda i, j: (i, 0)),
              pl.BlockSpec((LBN, D), lambda i, j: (j, 0)),
              pl.BlockSpec((LBM, LBN), lambda i, j: (i, j))],
    out_specs=pl.BlockSpec((1, 1), lambda i, j: (0, 0)),
    out_shape=jax.ShapeDtypeStruct((1, 1), jnp.float32),
    scratch_shapes=[pltpu.SMEM((4,), jnp.float32)],
    compiler_params=pltpu.CompilerParams(
        dimension_semantics=("arbitrary", "arbitrary")),
)


def kernel(feat_v, feat_u, neigh_v, neigh_u, target,
           Wv1, bv1, Wu1, bu1, a1, Wv2, bv2, Wu2, bu2, a2,
           Wv3, bv3, Wu3, bu3, Wd):
  iv = neigh_v.reshape(-1)
  iu = neigh_u.reshape(-1)

  aggv1, aggu1 = _sc_agg_mean(feat_u, iv, feat_v, iu)
  ve1, ue1 = _stage1_call(aggv1, aggu1, Wv1, Wu1,
                          bv1.reshape(1, D), bu1.reshape(1, D),
                          a1.reshape(1))
  aggv2, aggu2 = _sc_agg_mean(ue1, iv, ve1, iu)
  ve2, ue2, A = _stage2_call(aggv2, aggu2, Wv2, Wu2,
                             bv2.reshape(1, D), bu2.reshape(1, D),
                             a2.reshape(1), Wd)
  ve3, ue3 = _stage3_call(ve2, ue2, feat_v, feat_u, Wv3, Wu3,
                          bv3.reshape(1, D), bu3.reshape(1, D))
  sv, su = _sc_agg_self(ue3, iv, ve3, iu, ve3, ue3)
  loss = _loss_call(A, ue2, target)[0, 0]
  return ve2, ue2, sv, su, loss
